# Initial kernel scaffold; baseline (speedup 1.0000x reference)
import jax
import jax.numpy as jnp
from jax.experimental import pallas as pl


def kernel(bias_table, key_length):
    raise NotImplementedError("write your pallas kernel here")


# speedup vs baseline: 42.4507x; 42.4507x over previous
import functools
import math

import numpy as np
import jax
import jax.numpy as jnp
from jax import lax
from jax.experimental import pallas as pl
from jax.experimental.pallas import tpu as pltpu
from jax.experimental.pallas import tpu_sc as plsc

_NUM_BUCKETS = 32
_MAX_DISTANCE = 128
_HEADS = 16
_L = 2048
_PLEN = 2 * _L


def _bucket_host(rp: int) -> int:
    if rp < _NUM_BUCKETS // 2:
        return rp
    rp_f = np.float32(max(float(rp), 1.0))
    v = np.float32(np.log(rp_f / np.float32(16.0)))
    v = np.float32(v / np.float32(math.log(_MAX_DISTANCE / 16.0)))
    v = np.float32(v * np.float32(16.0))
    return min(16 + int(v), _NUM_BUCKETS - 1)


_BUCKETS_BY_RP = np.array([_bucket_host(r) for r in range(_L)], dtype=np.int64)
assert np.all(np.diff(_BUCKETS_BY_RP) >= 0), "bucket must be monotone in rp"
_THRESH = [int(np.argmax(_BUCKETS_BY_RP >= b)) for b in range(1, _NUM_BUCKETS)]
_T31 = _THRESH[-1]

_LANES = 16
_NCHUNK = _PLEN // _LANES
_CHUNK_LO = (_L - 1 - _T31 - 15 - 7) // 16
_CHUNK_HI = (_L - 1) // 16 + 1

_ROWS_PER_TEC = _L // 2
_GROUP = 16
_mesh = plsc.VectorSubcoreMesh(core_axis_name="c", subcore_axis_name="s")


@functools.partial(
    pl.kernel,
    out_type=jax.ShapeDtypeStruct((_HEADS * _L * _L,), jnp.float32),
    mesh=_mesh,
    scratch_types=[
        pltpu.VMEM((_NUM_BUCKETS, _HEADS), jnp.float32),
        pltpu.VMEM((8 * _PLEN,), jnp.float32),
        pltpu.SemaphoreType.DMA,
    ],
    compiler_params=pltpu.CompilerParams(needs_layout_passes=False),
)
def _sc_fill(table_hbm, out_hbm, tbl_v, q_v, sem):
    head = lax.axis_index("s")
    half = lax.axis_index("c")

    pltpu.sync_copy(table_hbm, tbl_v)
    lane = lax.broadcasted_iota(jnp.int32, (_LANES,), 0)
    hmask = lane == head
    tb = [
        jnp.sum(jnp.where(hmask, tbl_v[b, :], jnp.float32(0.0)))
        for b in range(_NUM_BUCKETS)
    ]
    deltas = [tb[b] - tb[b - 1] for b in range(1, _NUM_BUCKETS)]

    for s in range(8):
        base = s * _PLEN

        def splat31(q, carry, base=base):
            q_v[pl.ds(base + q * _LANES, _LANES)] = jnp.full(
                (_LANES,), tb[-1], dtype=jnp.float32)
            return carry

        def splat0(q, carry, base=base):
            q_v[pl.ds(base + q * _LANES, _LANES)] = jnp.full(
                (_LANES,), tb[0], dtype=jnp.float32)
            return carry

        def band(q, carry, base=base, s=s):
            rp = (_L - 1 - s) - (q * _LANES + lane)
            rp = jnp.maximum(rp, 0)
            val = jnp.full((_LANES,), tb[0], dtype=jnp.float32)
            for t, d in zip(_THRESH, deltas):
                val = val + jnp.where(rp >= t, d, jnp.float32(0.0))
            q_v[pl.ds(base + q * _LANES, _LANES)] = val
            return carry

        lax.fori_loop(0, _CHUNK_LO, splat31, 0)
        lax.fori_loop(_CHUNK_LO, _CHUNK_HI, band, 0)
        lax.fori_loop(_CHUNK_HI, _NCHUNK, splat0, 0)

    row0 = half * _ROWS_PER_TEC
    out_base = (head * _L + row0) * _L

    def emit(g, carry):
        copies = []
        for k in range(_GROUP):
            w = (_L - 1) - (row0 + g * _GROUP + k)
            s = (_L - 1 - k) % 8
            src = q_v.at[pl.ds(s * _PLEN + (w - s), _L)]
            dst = out_hbm.at[pl.ds(out_base + (g * _GROUP + k) * _L, _L)]
            copies.append(pltpu.make_async_copy(src, dst, sem))
        for cp in copies:
            cp.start()
        for cp in copies:
            cp.wait()
        return carry

    lax.fori_loop(0, _ROWS_PER_TEC // _GROUP, emit, 0)


def kernel(bias_table, key_length):
    del key_length
    flat = _sc_fill(bias_table)
    return flat.reshape(1, _HEADS, _L, _L)
